# Initial kernel scaffold; baseline (speedup 1.0000x reference)
import jax
import jax.numpy as jnp
from jax.experimental import pallas as pl


def kernel(x):
    raise NotImplementedError("write your pallas kernel here")


# speedup vs baseline: 1.1374x; 1.1374x over previous
import jax
import jax.numpy as jnp
from jax.experimental import pallas as pl


_B, _S, _K = 4, 8192, 256
_CHUNK = 1024


def _one_hot_kernel(x_ref, o_ref):
    x = x_ref[...]
    iota = jax.lax.broadcasted_iota(jnp.int32, (_B, _CHUNK, _K), 2)
    o_ref[...] = (x[:, :, None] == iota).astype(jnp.float32)


def kernel(x):
    grid = _S // _CHUNK
    return pl.pallas_call(
        _one_hot_kernel,
        grid=(grid,),
        in_specs=[pl.BlockSpec((_B, _CHUNK), lambda i: (0, i))],
        out_specs=pl.BlockSpec((_B, _CHUNK, _K), lambda i: (0, i, 0)),
        out_shape=jax.ShapeDtypeStruct((_B, _S, _K), jnp.float32),
    )(x)
